# Initial kernel scaffold; baseline (speedup 1.0000x reference)
import jax
import jax.numpy as jnp
from jax.experimental import pallas as pl


def kernel(x, edge_index, batch, W1, b1, g1, be1, W2, b2, g2, be2, lW1, lb1, lW2, lb2):
    raise NotImplementedError("write your pallas kernel here")


# speedup vs baseline: 27.8338x; 27.8338x over previous
import functools

import jax
import jax.numpy as jnp
from jax import lax
from jax.experimental import pallas as pl
from jax.experimental.pallas import tpu as pltpu
from jax.experimental.pallas import tpu_sc as plsc

N = 10000
E = 320000
F = 128
H = 64
G = 128
BN_EPS = 1e-5

NC = 2
NS = 16
LANES = 16
NW = NC * NS
CH = 125
NCHUNK = E // CH
CPW = NCHUNK // NW
RPT = N // NS

BLK = 500


def _sc_mesh():
    return plsc.VectorSubcoreMesh(core_axis_name="c", subcore_axis_name="s",
                                  num_cores=NC, num_subcores=NS)


@functools.partial(
    pl.kernel,
    out_type=jax.ShapeDtypeStruct((N, NC, LANES), jnp.float32),
    mesh=_sc_mesh(),
    scratch_types=[
        pltpu.VMEM((CPW, CH), jnp.int32),
        pltpu.VMEM((CH, LANES), jnp.float32),
        pltpu.VMEM((RPT, LANES), jnp.float32),
        pltpu.VMEM_SHARED((N, LANES), jnp.float32),
    ],
    compiler_params=pltpu.CompilerParams(use_tc_tiling_on_sc=False),
)
def _sc_degree(dst_hbm, out_hbm, didx, onesb, zbuf, deg_sh):
    cid = lax.axis_index("c")
    sid = lax.axis_index("s")
    wid = sid * NC + cid

    pltpu.sync_copy(dst_hbm.at[pl.ds(wid * CPW, CPW)], didx)

    def fill_ones(i, _):
        onesb[i] = jnp.ones((LANES,), jnp.float32)
        return 0

    lax.fori_loop(0, CH, fill_ones, 0)

    def fill_zeros(i, _):
        zbuf[i] = jnp.zeros((LANES,), jnp.float32)
        return 0

    lax.fori_loop(0, RPT, fill_zeros, 0)

    pltpu.sync_copy(zbuf, deg_sh.at[pl.ds(sid * RPT, RPT)])
    plsc.subcore_barrier()

    def body(j, _):
        pltpu.sync_copy(onesb, deg_sh.at[didx.at[j]], add=True)
        return 0

    lax.fori_loop(0, CPW, body, 0)
    plsc.subcore_barrier()

    pltpu.sync_copy(deg_sh.at[pl.ds(sid * RPT, RPT)],
                    out_hbm.at[pl.ds(sid * RPT, RPT), cid])


@functools.partial(
    pl.kernel,
    out_type=jax.ShapeDtypeStruct((N, NC, H), jnp.float32),
    mesh=_sc_mesh(),
    scratch_types=[
        pltpu.VMEM((CPW, CH), jnp.int32),
        pltpu.VMEM((CPW, CH), jnp.int32),
        pltpu.VMEM((CH, H), jnp.float32),
        pltpu.VMEM((CH, H), jnp.float32),
        pltpu.VMEM((RPT, H), jnp.float32),
        pltpu.VMEM_SHARED((N, H), jnp.float32),
        pltpu.SemaphoreType.DMA,
        pltpu.SemaphoreType.DMA,
    ],
    compiler_params=pltpu.CompilerParams(use_tc_tiling_on_sc=False),
)
def _sc_agg(src_hbm, dst_hbm, u_hbm, out_hbm,
            sidx, didx, rows_a, rows_b, zbuf, agg_sh, sem_a, sem_b):
    cid = lax.axis_index("c")
    sid = lax.axis_index("s")
    wid = sid * NC + cid

    pltpu.sync_copy(src_hbm.at[pl.ds(wid * CPW, CPW)], sidx)
    pltpu.sync_copy(dst_hbm.at[pl.ds(wid * CPW, CPW)], didx)

    def fill_zeros(i, _):
        for k in range(H // LANES):
            zbuf[i, pl.ds(k * LANES, LANES)] = jnp.zeros((LANES,), jnp.float32)
        return 0

    lax.fori_loop(0, RPT, fill_zeros, 0)

    pltpu.sync_copy(zbuf, agg_sh.at[pl.ds(sid * RPT, RPT)])
    plsc.subcore_barrier()

    pltpu.async_copy(u_hbm.at[sidx.at[0]], rows_a, sem_a)

    def body(j2, _):
        c0 = 2 * j2
        c1 = c0 + 1
        pltpu.async_copy(u_hbm.at[sidx.at[c1]], rows_b, sem_b)
        pltpu.make_async_copy(u_hbm.at[sidx.at[c0]], rows_a, sem_a).wait()
        pltpu.sync_copy(rows_a, agg_sh.at[didx.at[c0]], add=True)
        cn = jnp.minimum(c0 + 2, CPW - 1)
        pltpu.async_copy(u_hbm.at[sidx.at[cn]], rows_a, sem_a)
        pltpu.make_async_copy(u_hbm.at[sidx.at[c1]], rows_b, sem_b).wait()
        pltpu.sync_copy(rows_b, agg_sh.at[didx.at[c1]], add=True)
        return 0

    lax.fori_loop(0, CPW // 2, body, 0)
    pltpu.make_async_copy(u_hbm.at[sidx.at[CPW - 1]], rows_a, sem_a).wait()

    plsc.subcore_barrier()
    pltpu.sync_copy(agg_sh.at[pl.ds(sid * RPT, RPT)],
                    out_hbm.at[pl.ds(sid * RPT, RPT), cid])


_BN_S = 1.0 / (1.0 + BN_EPS) ** 0.5


def _t1_body(deg_ref, x_ref, w1_ref, u1_ref, dinv_ref):
    d = deg_ref[...]
    dsum = jnp.sum(d[:, :, 0:1], axis=1)
    dv = lax.rsqrt(dsum + 1.0)
    xw = jnp.dot(x_ref[...], w1_ref[...], preferred_element_type=jnp.float32)
    u1_ref[...] = xw * dv
    dinv_ref[...] = dv


def _t1(degp, x, w1):
    return pl.pallas_call(
        _t1_body,
        out_shape=[
            jax.ShapeDtypeStruct((N, H), jnp.float32),
            jax.ShapeDtypeStruct((N, 1), jnp.float32),
        ],
    )(degp, x, w1)


def _combine(p_ref, u_ref, dinv_ref, b_ref, g_ref, be_ref):
    dv = dinv_ref[...]
    agg = jnp.sum(p_ref[...], axis=1) + u_ref[...]
    h = dv * agg + b_ref[...]
    return jnp.maximum(h * (g_ref[...] * _BN_S) + be_ref[...], 0.0)


def _t2_body(p_ref, u_ref, dinv_ref, b_ref, g_ref, be_ref, w2_ref, u2_ref):
    h = _combine(p_ref, u_ref, dinv_ref, b_ref, g_ref, be_ref)
    u2_ref[...] = jnp.dot(h, w2_ref[...],
                          preferred_element_type=jnp.float32) * dinv_ref[...]


def _t2(p1, u1, dinv, b1, g1, be1, w2):
    return pl.pallas_call(
        _t2_body,
        out_shape=jax.ShapeDtypeStruct((N, H), jnp.float32),
    )(p1, u1, dinv, b1, g1, be1, w2)


def _t3_body(p_ref, u_ref, dinv_ref, b_ref, g_ref, be_ref, batch_ref,
             lw1_ref, lb1_ref, lw2_ref, lb2_ref, y_ref):
    h = _combine(p_ref, u_ref, dinv_ref, b_ref, g_ref, be_ref)
    bb = batch_ref[...]
    iota = lax.broadcasted_iota(jnp.int32, (G, N), 0)
    oh = (iota == bb).astype(jnp.float32)
    pool = jnp.dot(oh, h, preferred_element_type=jnp.float32)
    cnt = jnp.sum(oh, axis=1, keepdims=True)
    mean = pool / jnp.maximum(cnt, 1.0)
    t = jnp.maximum(
        jnp.dot(mean, lw1_ref[...],
                preferred_element_type=jnp.float32) + lb1_ref[...], 0.0)
    y_ref[...] = jnp.dot(t, lw2_ref[...],
                         preferred_element_type=jnp.float32) + lb2_ref[...]


def _t3(p2, u2, dinv, b2, g2, be2, batch2d, lw1, lb1, lw2, lb2):
    return pl.pallas_call(
        _t3_body,
        out_shape=jax.ShapeDtypeStruct((G, 2), jnp.float32),
    )(p2, u2, dinv, b2, g2, be2, batch2d, lw1, lb1, lw2, lb2)


def kernel(x, edge_index, batch, W1, b1, g1, be1, W2, b2, g2, be2,
           lW1, lb1, lW2, lb2):
    src2d = edge_index[0].reshape(NCHUNK, CH)
    dst2d = edge_index[1].reshape(NCHUNK, CH)

    degp = _sc_degree(dst2d)
    u1, dinv = _t1(degp, x, W1)
    p1 = _sc_agg(src2d, dst2d, u1)
    u2 = _t2(p1, u1, dinv, b1.reshape(1, H), g1.reshape(1, H),
             be1.reshape(1, H), W2)
    p2 = _sc_agg(src2d, dst2d, u2)
    y = _t3(p2, u2, dinv, b2.reshape(1, H), g2.reshape(1, H),
            be2.reshape(1, H), batch.reshape(1, N), lW1,
            lb1.reshape(1, H // 2), lW2, lb2.reshape(1, 2))
    return y


# speedup vs baseline: 32.9517x; 1.1839x over previous
import functools

import jax
import jax.numpy as jnp
from jax import lax
from jax.experimental import pallas as pl
from jax.experimental.pallas import tpu as pltpu
from jax.experimental.pallas import tpu_sc as plsc

N = 10000
E = 320000
F = 128
H = 64
G = 128
BN_EPS = 1e-5

NC = 2
NS = 16
LANES = 16
NW = NC * NS
CH = 128
NCHUNK = E // CH
CPW = NCHUNK // NW
XTRA = NCHUNK - CPW * NW
DW = 128
RPT = N // NS


def _sc_mesh():
    return plsc.VectorSubcoreMesh(core_axis_name="c", subcore_axis_name="s",
                                  num_cores=NC, num_subcores=NS)


@functools.partial(
    pl.kernel,
    out_type=jax.ShapeDtypeStruct((NW, N), jnp.float32),
    mesh=_sc_mesh(),
    scratch_types=[
        pltpu.VMEM((CPW + 1, CH), jnp.int32),
        pltpu.VMEM((N,), jnp.float32),
    ],
    compiler_params=pltpu.CompilerParams(use_tc_tiling_on_sc=False,
                                         needs_layout_passes=False),
)
def _sc_degree(dst_hbm, out_hbm, didx, degv):
    cid = lax.axis_index("c")
    sid = lax.axis_index("s")
    wid = sid * NC + cid

    pltpu.sync_copy(dst_hbm.at[pl.ds(wid * CPW, CPW)],
                    didx.at[pl.ds(0, CPW)])

    @pl.when(wid < XTRA)
    def _():
        pltpu.sync_copy(dst_hbm.at[pl.ds(NW * CPW + wid, 1)],
                        didx.at[pl.ds(CPW, 1)])

    def fill_zeros(i, _):
        degv[pl.ds(i * LANES, LANES)] = jnp.zeros((LANES,), jnp.float32)
        return 0

    lax.fori_loop(0, N // LANES, fill_zeros, 0)

    ones = jnp.ones((LANES,), jnp.float32)

    def body(r, _):
        for k in range(CH // LANES):
            idx = didx[r, pl.ds(k * LANES, LANES)]
            plsc.addupdate_scatter(degv, [idx], ones)
        return 0

    lax.fori_loop(0, CPW, body, 0)

    @pl.when(wid < XTRA)
    def _():
        for k in range(CH // LANES):
            idx = didx[CPW, pl.ds(k * LANES, LANES)]
            plsc.addupdate_scatter(degv, [idx], ones)

    pltpu.sync_copy(degv, out_hbm.at[wid])


@functools.partial(
    pl.kernel,
    out_type=jax.ShapeDtypeStruct((N, NC, H), jnp.float32),
    mesh=_sc_mesh(),
    scratch_types=[
        pltpu.VMEM((CPW + 1, CH), jnp.int32),
        pltpu.VMEM((CPW + 1, CH), jnp.int32),
        pltpu.VMEM((CH, H), jnp.float32),
        pltpu.VMEM((CH, H), jnp.float32),
        pltpu.VMEM((RPT, H), jnp.float32),
        pltpu.VMEM_SHARED((N, H), jnp.float32),
        pltpu.SemaphoreType.DMA,
        pltpu.SemaphoreType.DMA,
    ],
    compiler_params=pltpu.CompilerParams(use_tc_tiling_on_sc=False,
                                         needs_layout_passes=False),
)
def _sc_agg(src_hbm, dst_hbm, u_hbm, out_hbm,
            sidx, didx, rows_a, rows_b, zbuf, agg_sh, sem_a, sem_b):
    cid = lax.axis_index("c")
    sid = lax.axis_index("s")
    wid = sid * NC + cid

    pltpu.sync_copy(src_hbm.at[pl.ds(wid * CPW, CPW)],
                    sidx.at[pl.ds(0, CPW)])
    pltpu.sync_copy(dst_hbm.at[pl.ds(wid * CPW, CPW)],
                    didx.at[pl.ds(0, CPW)])

    @pl.when(wid < XTRA)
    def _():
        pltpu.sync_copy(src_hbm.at[pl.ds(NW * CPW + wid, 1)],
                        sidx.at[pl.ds(CPW, 1)])
        pltpu.sync_copy(dst_hbm.at[pl.ds(NW * CPW + wid, 1)],
                        didx.at[pl.ds(CPW, 1)])

    def fill_zeros(i, _):
        for k in range(H // LANES):
            zbuf[i, pl.ds(k * LANES, LANES)] = jnp.zeros((LANES,), jnp.float32)
        return 0

    lax.fori_loop(0, RPT, fill_zeros, 0)

    pltpu.sync_copy(zbuf, agg_sh.at[pl.ds(sid * RPT, RPT)])
    plsc.subcore_barrier()

    pltpu.async_copy(u_hbm.at[sidx.at[0]], rows_a, sem_a)

    def body(j2, _):
        c0 = 2 * j2
        c1 = c0 + 1
        pltpu.async_copy(u_hbm.at[sidx.at[c1]], rows_b, sem_b)
        pltpu.make_async_copy(u_hbm.at[sidx.at[c0]], rows_a, sem_a).wait()
        pltpu.sync_copy(rows_a, agg_sh.at[didx.at[c0]], add=True)
        cn = jnp.minimum(c0 + 2, CPW - 1)
        pltpu.async_copy(u_hbm.at[sidx.at[cn]], rows_a, sem_a)
        pltpu.make_async_copy(u_hbm.at[sidx.at[c1]], rows_b, sem_b).wait()
        pltpu.sync_copy(rows_b, agg_sh.at[didx.at[c1]], add=True)
        return 0

    lax.fori_loop(0, CPW // 2, body, 0)
    pltpu.make_async_copy(u_hbm.at[sidx.at[CPW - 1]], rows_a, sem_a).wait()

    @pl.when(wid < XTRA)
    def _():
        pltpu.sync_copy(u_hbm.at[sidx.at[CPW]], rows_a)
        pltpu.sync_copy(rows_a, agg_sh.at[didx.at[CPW]], add=True)

    plsc.subcore_barrier()
    pltpu.sync_copy(agg_sh.at[pl.ds(sid * RPT, RPT)],
                    out_hbm.at[pl.ds(sid * RPT, RPT), cid])


_BN_S = 1.0 / (1.0 + BN_EPS) ** 0.5


def _t1_body(deg_ref, x_ref, w1_ref, u1_ref, dinv_ref):
    d = deg_ref[...]
    dsum = lax.dot_general(d, jnp.ones((NW, 1), jnp.float32),
                           (((0,), (0,)), ((), ())),
                           preferred_element_type=jnp.float32)
    dv = lax.rsqrt(dsum + 1.0)
    xw = jnp.dot(x_ref[...], w1_ref[...], preferred_element_type=jnp.float32)
    u1_ref[...] = xw * dv
    dinv_ref[...] = dv


def _t1(degp, x, w1):
    return pl.pallas_call(
        _t1_body,
        out_shape=[
            jax.ShapeDtypeStruct((N, H), jnp.float32),
            jax.ShapeDtypeStruct((N, 1), jnp.float32),
        ],
    )(degp, x, w1)


def _combine(p_ref, u_ref, dinv_ref, b_ref, g_ref, be_ref):
    dv = dinv_ref[...]
    agg = jnp.sum(p_ref[...], axis=1) + u_ref[...]
    h = dv * agg + b_ref[...]
    return jnp.maximum(h * (g_ref[...] * _BN_S) + be_ref[...], 0.0)


def _t2_body(p_ref, u_ref, dinv_ref, b_ref, g_ref, be_ref, w2_ref, u2_ref):
    h = _combine(p_ref, u_ref, dinv_ref, b_ref, g_ref, be_ref)
    u2_ref[...] = jnp.dot(h, w2_ref[...],
                          preferred_element_type=jnp.float32) * dinv_ref[...]


def _t2(p1, u1, dinv, b1, g1, be1, w2):
    return pl.pallas_call(
        _t2_body,
        out_shape=jax.ShapeDtypeStruct((N, H), jnp.float32),
    )(p1, u1, dinv, b1, g1, be1, w2)


def _t3_body(p_ref, u_ref, dinv_ref, b_ref, g_ref, be_ref, batch_ref,
             lw1_ref, lb1_ref, lw2_ref, lb2_ref, y_ref):
    h = _combine(p_ref, u_ref, dinv_ref, b_ref, g_ref, be_ref)
    bb = batch_ref[...]
    iota = lax.broadcasted_iota(jnp.int32, (G, N), 0)
    oh = (iota == bb).astype(jnp.float32)
    pool = jnp.dot(oh, h, preferred_element_type=jnp.float32)
    cnt = jnp.sum(oh, axis=1, keepdims=True)
    mean = pool / jnp.maximum(cnt, 1.0)
    t = jnp.maximum(
        jnp.dot(mean, lw1_ref[...],
                preferred_element_type=jnp.float32) + lb1_ref[...], 0.0)
    y_ref[...] = jnp.dot(t, lw2_ref[...],
                         preferred_element_type=jnp.float32) + lb2_ref[...]


def _t3(p2, u2, dinv, b2, g2, be2, batch2d, lw1, lb1, lw2, lb2):
    return pl.pallas_call(
        _t3_body,
        out_shape=jax.ShapeDtypeStruct((G, 2), jnp.float32),
    )(p2, u2, dinv, b2, g2, be2, batch2d, lw1, lb1, lw2, lb2)


def kernel(x, edge_index, batch, W1, b1, g1, be1, W2, b2, g2, be2,
           lW1, lb1, lW2, lb2):
    src2d = edge_index[0].reshape(NCHUNK, CH)
    dst2d = edge_index[1].reshape(NCHUNK, CH)

    degp = _sc_degree(dst2d)
    u1, dinv = _t1(degp, x, W1)
    p1 = _sc_agg(src2d, dst2d, u1)
    u2 = _t2(p1, u1, dinv, b1.reshape(1, H), g1.reshape(1, H),
             be1.reshape(1, H), W2)
    p2 = _sc_agg(src2d, dst2d, u2)
    y = _t3(p2, u2, dinv, b2.reshape(1, H), g2.reshape(1, H),
            be2.reshape(1, H), batch.reshape(1, N), lW1,
            lb1.reshape(1, H // 2), lW2, lb2.reshape(1, 2))
    return y


# speedup vs baseline: 47.3433x; 1.4367x over previous
import functools

import jax
import jax.numpy as jnp
from jax import lax
from jax.experimental import pallas as pl
from jax.experimental.pallas import tpu as pltpu
from jax.experimental.pallas import tpu_sc as plsc

N = 10000
E = 320000
F = 128
H = 64
G = 128
BN_EPS = 1e-5

NC = 2
NS = 16
LANES = 16
NW = NC * NS
CH = 128
NCHUNK = E // CH
CPW = NCHUNK // NW
XTRA = NCHUNK - CPW * NW
DW = 128
RPT = N // NS


def _sc_mesh():
    return plsc.VectorSubcoreMesh(core_axis_name="c", subcore_axis_name="s",
                                  num_cores=NC, num_subcores=NS)


@functools.partial(
    pl.kernel,
    out_type=jax.ShapeDtypeStruct((NW, N), jnp.float32),
    mesh=_sc_mesh(),
    scratch_types=[
        pltpu.VMEM((CPW + 1, CH), jnp.int32),
        pltpu.VMEM((N,), jnp.float32),
    ],
    compiler_params=pltpu.CompilerParams(use_tc_tiling_on_sc=False,
                                         needs_layout_passes=False),
)
def _sc_degree(e_hbm, out_hbm, didx, degv):
    cid = lax.axis_index("c")
    sid = lax.axis_index("s")
    wid = sid * NC + cid

    pltpu.sync_copy(e_hbm.at[1, pl.ds(wid * CPW, CPW)],
                    didx.at[pl.ds(0, CPW)])

    @pl.when(wid < XTRA)
    def _():
        pltpu.sync_copy(e_hbm.at[1, pl.ds(NW * CPW + wid, 1)],
                        didx.at[pl.ds(CPW, 1)])

    def fill_zeros(i, _):
        degv[pl.ds(i * LANES, LANES)] = jnp.zeros((LANES,), jnp.float32)
        return 0

    lax.fori_loop(0, N // LANES, fill_zeros, 0)

    ones = jnp.ones((LANES,), jnp.float32)

    def body(r, _):
        for k in range(CH // LANES):
            idx = didx[r, pl.ds(k * LANES, LANES)]
            plsc.addupdate_scatter(degv, [idx], ones)
        return 0

    lax.fori_loop(0, CPW, body, 0)

    @pl.when(wid < XTRA)
    def _():
        for k in range(CH // LANES):
            idx = didx[CPW, pl.ds(k * LANES, LANES)]
            plsc.addupdate_scatter(degv, [idx], ones)

    pltpu.sync_copy(degv, out_hbm.at[wid])


@functools.partial(
    pl.kernel,
    out_type=jax.ShapeDtypeStruct((N, NC * H), jnp.float32),
    mesh=_sc_mesh(),
    scratch_types=[
        pltpu.VMEM((CPW + 1, CH), jnp.int32),
        pltpu.VMEM((CPW + 1, CH), jnp.int32),
        pltpu.VMEM((CH, H), jnp.float32),
        pltpu.VMEM((CH, H), jnp.float32),
        pltpu.VMEM((RPT, H), jnp.float32),
        pltpu.VMEM_SHARED((N, H), jnp.float32),
        pltpu.SemaphoreType.DMA,
        pltpu.SemaphoreType.DMA,
    ],
    compiler_params=pltpu.CompilerParams(use_tc_tiling_on_sc=False,
                                         needs_layout_passes=False),
)
def _sc_agg(e_hbm, u_hbm, out_hbm,
            sidx, didx, rows_a, rows_b, zbuf, agg_sh, sem_a, sem_b):
    cid = lax.axis_index("c")
    sid = lax.axis_index("s")
    wid = sid * NC + cid

    pltpu.sync_copy(e_hbm.at[0, pl.ds(wid * CPW, CPW)],
                    sidx.at[pl.ds(0, CPW)])
    pltpu.sync_copy(e_hbm.at[1, pl.ds(wid * CPW, CPW)],
                    didx.at[pl.ds(0, CPW)])

    @pl.when(wid < XTRA)
    def _():
        pltpu.sync_copy(e_hbm.at[0, pl.ds(NW * CPW + wid, 1)],
                        sidx.at[pl.ds(CPW, 1)])
        pltpu.sync_copy(e_hbm.at[1, pl.ds(NW * CPW + wid, 1)],
                        didx.at[pl.ds(CPW, 1)])

    def fill_zeros(i, _):
        for k in range(H // LANES):
            zbuf[i, pl.ds(k * LANES, LANES)] = jnp.zeros((LANES,), jnp.float32)
        return 0

    lax.fori_loop(0, RPT, fill_zeros, 0)

    pltpu.sync_copy(zbuf, agg_sh.at[pl.ds(sid * RPT, RPT)])
    plsc.subcore_barrier()

    pltpu.async_copy(u_hbm.at[sidx.at[0]], rows_a, sem_a)

    def body(j2, _):
        c0 = 2 * j2
        c1 = c0 + 1
        pltpu.async_copy(u_hbm.at[sidx.at[c1]], rows_b, sem_b)
        pltpu.make_async_copy(u_hbm.at[sidx.at[c0]], rows_a, sem_a).wait()
        pltpu.sync_copy(rows_a, agg_sh.at[didx.at[c0]], add=True)
        cn = jnp.minimum(c0 + 2, CPW - 1)
        pltpu.async_copy(u_hbm.at[sidx.at[cn]], rows_a, sem_a)
        pltpu.make_async_copy(u_hbm.at[sidx.at[c1]], rows_b, sem_b).wait()
        pltpu.sync_copy(rows_b, agg_sh.at[didx.at[c1]], add=True)
        return 0

    lax.fori_loop(0, CPW // 2, body, 0)
    pltpu.make_async_copy(u_hbm.at[sidx.at[CPW - 1]], rows_a, sem_a).wait()

    @pl.when(wid < XTRA)
    def _():
        pltpu.sync_copy(u_hbm.at[sidx.at[CPW]], rows_a)
        pltpu.sync_copy(rows_a, agg_sh.at[didx.at[CPW]], add=True)

    plsc.subcore_barrier()
    pltpu.sync_copy(agg_sh.at[pl.ds(sid * RPT, RPT)],
                    out_hbm.at[pl.ds(sid * RPT, RPT), pl.ds(cid * H, H)])


_BN_S = 1.0 / (1.0 + BN_EPS) ** 0.5


def _t1_body(deg_ref, x_ref, w1_ref, u1_ref, dinv_ref):
    d = deg_ref[...]
    dsum = lax.dot_general(d, jnp.ones((NW, 1), jnp.float32),
                           (((0,), (0,)), ((), ())),
                           preferred_element_type=jnp.float32)
    dv = lax.rsqrt(dsum + 1.0)
    xw = jnp.dot(x_ref[...], w1_ref[...], preferred_element_type=jnp.float32)
    u1_ref[...] = xw * dv
    dinv_ref[...] = dv


def _t1(degp, x, w1):
    return pl.pallas_call(
        _t1_body,
        out_shape=[
            jax.ShapeDtypeStruct((N, H), jnp.float32),
            jax.ShapeDtypeStruct((N, 1), jnp.float32),
        ],
    )(degp, x, w1)


def _combine(p_ref, u_ref, dinv_ref, b_ref, g_ref, be_ref):
    dv = dinv_ref[...]
    p = p_ref[...]
    agg = p[:, 0:H] + p[:, H:NC * H] + u_ref[...]
    h = dv * agg + b_ref[...]
    return jnp.maximum(h * (g_ref[...] * _BN_S) + be_ref[...], 0.0)


def _t2_body(p_ref, u_ref, dinv_ref, b_ref, g_ref, be_ref, w2_ref, u2_ref):
    h = _combine(p_ref, u_ref, dinv_ref, b_ref, g_ref, be_ref)
    u2_ref[...] = jnp.dot(h, w2_ref[...],
                          preferred_element_type=jnp.float32) * dinv_ref[...]


def _t2(p1, u1, dinv, b1, g1, be1, w2):
    return pl.pallas_call(
        _t2_body,
        out_shape=jax.ShapeDtypeStruct((N, H), jnp.float32),
    )(p1, u1, dinv, b1, g1, be1, w2)


def _t3_body(p_ref, u_ref, dinv_ref, b_ref, g_ref, be_ref, batch_ref,
             lw1_ref, lb1_ref, lw2_ref, lb2_ref, y_ref):
    h = _combine(p_ref, u_ref, dinv_ref, b_ref, g_ref, be_ref)
    bb = batch_ref[...]
    iota = lax.broadcasted_iota(jnp.int32, (G, N), 0)
    oh = (iota == bb).astype(jnp.float32)
    pool = jnp.dot(oh, h, preferred_element_type=jnp.float32)
    cnt = jnp.sum(oh, axis=1, keepdims=True)
    mean = pool / jnp.maximum(cnt, 1.0)
    t = jnp.maximum(
        jnp.dot(mean, lw1_ref[...],
                preferred_element_type=jnp.float32) + lb1_ref[...], 0.0)
    y_ref[...] = jnp.dot(t, lw2_ref[...],
                         preferred_element_type=jnp.float32) + lb2_ref[...]


def _t3(p2, u2, dinv, b2, g2, be2, batch2d, lw1, lb1, lw2, lb2):
    return pl.pallas_call(
        _t3_body,
        out_shape=jax.ShapeDtypeStruct((G, 2), jnp.float32),
    )(p2, u2, dinv, b2, g2, be2, batch2d, lw1, lb1, lw2, lb2)


def kernel(x, edge_index, batch, W1, b1, g1, be1, W2, b2, g2, be2,
           lW1, lb1, lW2, lb2):
    e3d = edge_index.reshape(2, NCHUNK, CH)

    degp = _sc_degree(e3d)
    u1, dinv = _t1(degp, x, W1)
    p1 = _sc_agg(e3d, u1)
    u2 = _t2(p1, u1, dinv, b1.reshape(1, H), g1.reshape(1, H),
             be1.reshape(1, H), W2)
    p2 = _sc_agg(e3d, u2)
    y = _t3(p2, u2, dinv, b2.reshape(1, H), g2.reshape(1, H),
            be2.reshape(1, H), batch.reshape(1, N), lW1,
            lb1.reshape(1, H // 2), lW2, lb2.reshape(1, 2))
    return y


# speedup vs baseline: 51.3399x; 1.0844x over previous
import functools

import jax
import jax.numpy as jnp
from jax import lax
from jax.experimental import pallas as pl
from jax.experimental.pallas import tpu as pltpu
from jax.experimental.pallas import tpu_sc as plsc

N = 10000
E = 320000
F = 128
H = 64
G = 128
BN_EPS = 1e-5

NC = 2
NS = 16
LANES = 16
NW = NC * NS
CH = 128
NCHUNK = E // CH
CPW = NCHUNK // NW
XTRA = NCHUNK - CPW * NW
DW = 128
RPT = N // NS
ZR = 125
NBUF = 4


def _sc_mesh():
    return plsc.VectorSubcoreMesh(core_axis_name="c", subcore_axis_name="s",
                                  num_cores=NC, num_subcores=NS)


@functools.partial(
    pl.kernel,
    out_type=jax.ShapeDtypeStruct((NW, N), jnp.float32),
    mesh=_sc_mesh(),
    scratch_types=[
        pltpu.VMEM((CPW + 1, CH), jnp.int32),
        pltpu.VMEM((N,), jnp.float32),
    ],
    compiler_params=pltpu.CompilerParams(use_tc_tiling_on_sc=False,
                                         needs_layout_passes=False),
)
def _sc_degree(e_hbm, out_hbm, didx, degv):
    cid = lax.axis_index("c")
    sid = lax.axis_index("s")
    wid = sid * NC + cid

    pltpu.sync_copy(e_hbm.at[1, pl.ds(wid * CPW, CPW)],
                    didx.at[pl.ds(0, CPW)])

    @pl.when(wid < XTRA)
    def _():
        pltpu.sync_copy(e_hbm.at[1, pl.ds(NW * CPW + wid, 1)],
                        didx.at[pl.ds(CPW, 1)])

    def fill_zeros(i, _):
        degv[pl.ds(i * LANES, LANES)] = jnp.zeros((LANES,), jnp.float32)
        return 0

    lax.fori_loop(0, N // LANES, fill_zeros, 0)

    ones = jnp.ones((LANES,), jnp.float32)

    def body(r, _):
        for k in range(CH // LANES):
            idx = didx[r, pl.ds(k * LANES, LANES)]
            plsc.addupdate_scatter(degv, [idx], ones)
        return 0

    lax.fori_loop(0, CPW, body, 0)

    @pl.when(wid < XTRA)
    def _():
        for k in range(CH // LANES):
            idx = didx[CPW, pl.ds(k * LANES, LANES)]
            plsc.addupdate_scatter(degv, [idx], ones)

    pltpu.sync_copy(degv, out_hbm.at[wid])


@functools.partial(
    pl.kernel,
    out_type=jax.ShapeDtypeStruct((N, NC * H), jnp.float32),
    mesh=_sc_mesh(),
    scratch_types=[
        pltpu.VMEM((CPW + 1, CH), jnp.int32),
        pltpu.VMEM((CPW + 1, CH), jnp.int32),
        [pltpu.VMEM((CH, H), jnp.float32)] * NBUF,
        pltpu.VMEM((ZR, H), jnp.float32),
        pltpu.VMEM_SHARED((N, H), jnp.float32),
        [pltpu.SemaphoreType.DMA] * NBUF,
        [pltpu.SemaphoreType.DMA] * NBUF,
    ],
    compiler_params=pltpu.CompilerParams(use_tc_tiling_on_sc=False,
                                         needs_layout_passes=False),
)
def _sc_agg(e_hbm, u_hbm, out_hbm,
            sidx, didx, rows, zbuf, agg_sh, gs, ss):
    cid = lax.axis_index("c")
    sid = lax.axis_index("s")
    wid = sid * NC + cid

    pltpu.sync_copy(e_hbm.at[0, pl.ds(wid * CPW, CPW)],
                    sidx.at[pl.ds(0, CPW)])
    pltpu.sync_copy(e_hbm.at[1, pl.ds(wid * CPW, CPW)],
                    didx.at[pl.ds(0, CPW)])

    @pl.when(wid < XTRA)
    def _():
        pltpu.sync_copy(e_hbm.at[0, pl.ds(NW * CPW + wid, 1)],
                        sidx.at[pl.ds(CPW, 1)])
        pltpu.sync_copy(e_hbm.at[1, pl.ds(NW * CPW + wid, 1)],
                        didx.at[pl.ds(CPW, 1)])

    def fill_zeros(i, _):
        for k in range(H // LANES):
            zbuf[i, pl.ds(k * LANES, LANES)] = jnp.zeros((LANES,), jnp.float32)
        return 0

    lax.fori_loop(0, ZR, fill_zeros, 0)

    for z in range(RPT // ZR):
        pltpu.sync_copy(zbuf, agg_sh.at[pl.ds(sid * RPT + z * ZR, ZR)])
    plsc.subcore_barrier()

    for b in range(NBUF):
        pltpu.async_copy(u_hbm.at[sidx.at[b]], rows[b], gs[b])

    def body(j2, _):
        for b in range(NBUF):
            c = NBUF * j2 + b
            pltpu.make_async_copy(u_hbm.at[sidx.at[c]], rows[b], gs[b]).wait()
            pltpu.async_copy(rows[b], agg_sh.at[didx.at[c]], ss[b], add=True)
        for b in range(NBUF):
            cp = jnp.minimum(NBUF * j2 + NBUF + b, CPW - 1)
            pltpu.make_async_copy(rows[b], agg_sh.at[didx.at[0]], ss[b]).wait()
            pltpu.async_copy(u_hbm.at[sidx.at[cp]], rows[b], gs[b])
        return 0

    lax.fori_loop(0, CPW // NBUF, body, 0)
    for b in range(NBUF):
        c = (CPW // NBUF) * NBUF + b
        pltpu.make_async_copy(
            u_hbm.at[sidx.at[min(c, CPW - 1)]], rows[b], gs[b]).wait()
        if c < CPW:
            pltpu.sync_copy(rows[b], agg_sh.at[didx.at[c]], add=True)

    @pl.when(wid < XTRA)
    def _():
        pltpu.sync_copy(u_hbm.at[sidx.at[CPW]], rows[NBUF - 1])
        pltpu.sync_copy(rows[NBUF - 1], agg_sh.at[didx.at[CPW]], add=True)

    plsc.subcore_barrier()
    pltpu.sync_copy(agg_sh.at[pl.ds(sid * RPT, RPT)],
                    out_hbm.at[pl.ds(sid * RPT, RPT), pl.ds(cid * H, H)])


_BN_S = 1.0 / (1.0 + BN_EPS) ** 0.5


def _t1_body(deg_ref, x_ref, w1_ref, u1_ref, dinv_ref):
    d = deg_ref[...]
    dsum = lax.dot_general(d, jnp.ones((NW, 1), jnp.float32),
                           (((0,), (0,)), ((), ())),
                           preferred_element_type=jnp.float32)
    dv = lax.rsqrt(dsum + 1.0)
    xw = jnp.dot(x_ref[...], w1_ref[...], preferred_element_type=jnp.float32)
    u1_ref[...] = xw * dv
    dinv_ref[...] = dv


def _t1(degp, x, w1):
    return pl.pallas_call(
        _t1_body,
        out_shape=[
            jax.ShapeDtypeStruct((N, H), jnp.float32),
            jax.ShapeDtypeStruct((N, 1), jnp.float32),
        ],
    )(degp, x, w1)


def _combine(p_ref, u_ref, dinv_ref, b_ref, g_ref, be_ref):
    dv = dinv_ref[...]
    p = p_ref[...]
    agg = p[:, 0:H] + p[:, H:NC * H] + u_ref[...]
    h = dv * agg + b_ref[...]
    return jnp.maximum(h * (g_ref[...] * _BN_S) + be_ref[...], 0.0)


def _t2_body(p_ref, u_ref, dinv_ref, b_ref, g_ref, be_ref, w2_ref, u2_ref):
    h = _combine(p_ref, u_ref, dinv_ref, b_ref, g_ref, be_ref)
    u2_ref[...] = jnp.dot(h, w2_ref[...],
                          preferred_element_type=jnp.float32) * dinv_ref[...]


def _t2(p1, u1, dinv, b1, g1, be1, w2):
    return pl.pallas_call(
        _t2_body,
        out_shape=jax.ShapeDtypeStruct((N, H), jnp.float32),
    )(p1, u1, dinv, b1, g1, be1, w2)


def _t3_body(p_ref, u_ref, dinv_ref, b_ref, g_ref, be_ref, batch_ref,
             lw1_ref, lb1_ref, lw2_ref, lb2_ref, y_ref):
    h = _combine(p_ref, u_ref, dinv_ref, b_ref, g_ref, be_ref)
    bb = batch_ref[...]
    iota = lax.broadcasted_iota(jnp.int32, (G, N), 0)
    oh = (iota == bb).astype(jnp.float32)
    pool = jnp.dot(oh, h, preferred_element_type=jnp.float32)
    cnt = jnp.sum(oh, axis=1, keepdims=True)
    mean = pool / jnp.maximum(cnt, 1.0)
    t = jnp.maximum(
        jnp.dot(mean, lw1_ref[...],
                preferred_element_type=jnp.float32) + lb1_ref[...], 0.0)
    y_ref[...] = jnp.dot(t, lw2_ref[...],
                         preferred_element_type=jnp.float32) + lb2_ref[...]


def _t3(p2, u2, dinv, b2, g2, be2, batch2d, lw1, lb1, lw2, lb2):
    return pl.pallas_call(
        _t3_body,
        out_shape=jax.ShapeDtypeStruct((G, 2), jnp.float32),
    )(p2, u2, dinv, b2, g2, be2, batch2d, lw1, lb1, lw2, lb2)


def kernel(x, edge_index, batch, W1, b1, g1, be1, W2, b2, g2, be2,
           lW1, lb1, lW2, lb2):
    e3d = edge_index.reshape(2, NCHUNK, CH)

    degp = _sc_degree(e3d)
    u1, dinv = _t1(degp, x, W1)
    p1 = _sc_agg(e3d, u1)
    u2 = _t2(p1, u1, dinv, b1.reshape(1, H), g1.reshape(1, H),
             be1.reshape(1, H), W2)
    p2 = _sc_agg(e3d, u2)
    y = _t3(p2, u2, dinv, b2.reshape(1, H), g2.reshape(1, H),
            be2.reshape(1, H), batch.reshape(1, N), lW1,
            lb1.reshape(1, H // 2), lW2, lb2.reshape(1, 2))
    return y


# speedup vs baseline: 52.8518x; 1.0294x over previous
import functools

import jax
import jax.numpy as jnp
from jax import lax
from jax.experimental import pallas as pl
from jax.experimental.pallas import tpu as pltpu
from jax.experimental.pallas import tpu_sc as plsc

N = 10000
E = 320000
F = 128
H = 64
G = 128
BN_EPS = 1e-5

NC = 2
NS = 16
LANES = 16
NW = NC * NS
CH = 128
NCHUNK = E // CH
CPW = NCHUNK // NW
XTRA = NCHUNK - CPW * NW
DW = 128
RPT = N // NS
ZR = 125
NBUF = 6


def _sc_mesh():
    return plsc.VectorSubcoreMesh(core_axis_name="c", subcore_axis_name="s",
                                  num_cores=NC, num_subcores=NS)


@functools.partial(
    pl.kernel,
    out_type=jax.ShapeDtypeStruct((NW, N), jnp.float32),
    mesh=_sc_mesh(),
    scratch_types=[
        pltpu.VMEM((CPW + 1, CH), jnp.int32),
        pltpu.VMEM((N,), jnp.float32),
    ],
    compiler_params=pltpu.CompilerParams(use_tc_tiling_on_sc=False,
                                         needs_layout_passes=False),
)
def _sc_degree(e_hbm, out_hbm, didx, degv):
    cid = lax.axis_index("c")
    sid = lax.axis_index("s")
    wid = sid * NC + cid

    pltpu.sync_copy(e_hbm.at[1, pl.ds(wid * CPW, CPW)],
                    didx.at[pl.ds(0, CPW)])

    @pl.when(wid < XTRA)
    def _():
        pltpu.sync_copy(e_hbm.at[1, pl.ds(NW * CPW + wid, 1)],
                        didx.at[pl.ds(CPW, 1)])

    def fill_zeros(i, _):
        degv[pl.ds(i * LANES, LANES)] = jnp.zeros((LANES,), jnp.float32)
        return 0

    lax.fori_loop(0, N // LANES, fill_zeros, 0)

    ones = jnp.ones((LANES,), jnp.float32)

    def body(r, _):
        for k in range(CH // LANES):
            idx = didx[r, pl.ds(k * LANES, LANES)]
            plsc.addupdate_scatter(degv, [idx], ones)
        return 0

    lax.fori_loop(0, CPW, body, 0)

    @pl.when(wid < XTRA)
    def _():
        for k in range(CH // LANES):
            idx = didx[CPW, pl.ds(k * LANES, LANES)]
            plsc.addupdate_scatter(degv, [idx], ones)

    pltpu.sync_copy(degv, out_hbm.at[wid])


@functools.partial(
    pl.kernel,
    out_type=jax.ShapeDtypeStruct((N, NC * H), jnp.float32),
    mesh=_sc_mesh(),
    scratch_types=[
        pltpu.VMEM((CPW + 1, CH), jnp.int32),
        pltpu.VMEM((CPW + 1, CH), jnp.int32),
        [pltpu.VMEM((CH, H), jnp.float32)] * NBUF,
        pltpu.VMEM((ZR, H), jnp.float32),
        pltpu.VMEM_SHARED((N, H), jnp.float32),
        [pltpu.SemaphoreType.DMA] * NBUF,
        [pltpu.SemaphoreType.DMA] * NBUF,
    ],
    compiler_params=pltpu.CompilerParams(use_tc_tiling_on_sc=False,
                                         needs_layout_passes=False),
)
def _sc_agg(e_hbm, u_hbm, out_hbm,
            sidx, didx, rows, zbuf, agg_sh, gs, ss):
    cid = lax.axis_index("c")
    sid = lax.axis_index("s")
    wid = sid * NC + cid

    cps = pltpu.async_copy(e_hbm.at[0, pl.ds(wid * CPW, CPW)],
                           sidx.at[pl.ds(0, CPW)], gs[0])
    cpd = pltpu.async_copy(e_hbm.at[1, pl.ds(wid * CPW, CPW)],
                           didx.at[pl.ds(0, CPW)], gs[1])

    @pl.when(wid < XTRA)
    def _():
        pltpu.sync_copy(e_hbm.at[0, pl.ds(NW * CPW + wid, 1)],
                        sidx.at[pl.ds(CPW, 1)])
        pltpu.sync_copy(e_hbm.at[1, pl.ds(NW * CPW + wid, 1)],
                        didx.at[pl.ds(CPW, 1)])

    def fill_zeros(i, _):
        for k in range(H // LANES):
            zbuf[i, pl.ds(k * LANES, LANES)] = jnp.zeros((LANES,), jnp.float32)
        return 0

    lax.fori_loop(0, ZR, fill_zeros, 0)

    for z in range(RPT // ZR):
        pltpu.sync_copy(zbuf, agg_sh.at[pl.ds(sid * RPT + z * ZR, ZR)])
    cps.wait()
    cpd.wait()
    plsc.subcore_barrier()

    for b in range(NBUF):
        pltpu.async_copy(u_hbm.at[sidx.at[b]], rows[b], gs[b])

    def body(j2, _):
        for b in range(NBUF):
            c = NBUF * j2 + b
            pltpu.make_async_copy(u_hbm.at[sidx.at[c]], rows[b], gs[b]).wait()
            pltpu.async_copy(rows[b], agg_sh.at[didx.at[c]], ss[b], add=True)
        for b in range(NBUF):
            cp = jnp.minimum(NBUF * j2 + NBUF + b, CPW - 1)
            pltpu.make_async_copy(rows[b], agg_sh.at[didx.at[0]], ss[b]).wait()
            pltpu.async_copy(u_hbm.at[sidx.at[cp]], rows[b], gs[b])
        return 0

    lax.fori_loop(0, CPW // NBUF, body, 0)
    for b in range(NBUF):
        c = (CPW // NBUF) * NBUF + b
        pltpu.make_async_copy(
            u_hbm.at[sidx.at[min(c, CPW - 1)]], rows[b], gs[b]).wait()
        if c < CPW:
            pltpu.sync_copy(rows[b], agg_sh.at[didx.at[c]], add=True)

    @pl.when(wid < XTRA)
    def _():
        pltpu.sync_copy(u_hbm.at[sidx.at[CPW]], rows[NBUF - 1])
        pltpu.sync_copy(rows[NBUF - 1], agg_sh.at[didx.at[CPW]], add=True)

    plsc.subcore_barrier()
    pltpu.sync_copy(agg_sh.at[pl.ds(sid * RPT, RPT)],
                    out_hbm.at[pl.ds(sid * RPT, RPT), pl.ds(cid * H, H)])


_BN_S = 1.0 / (1.0 + BN_EPS) ** 0.5


def _t1a_body(x_ref, w1_ref, xw_ref):
    xw_ref[...] = jnp.dot(x_ref[...], w1_ref[...],
                          preferred_element_type=jnp.float32)


def _t1a(x, w1):
    return pl.pallas_call(
        _t1a_body,
        out_shape=jax.ShapeDtypeStruct((N, H), jnp.float32),
    )(x, w1)


def _t1b_body(deg_ref, xw_ref, u1_ref, dinv_ref):
    d = deg_ref[...]
    dsum = lax.dot_general(d, jnp.ones((NW, 1), jnp.float32),
                           (((0,), (0,)), ((), ())),
                           preferred_element_type=jnp.float32)
    dv = lax.rsqrt(dsum + 1.0)
    u1_ref[...] = xw_ref[...] * dv
    dinv_ref[...] = dv


def _t1b(degp, xw):
    return pl.pallas_call(
        _t1b_body,
        out_shape=[
            jax.ShapeDtypeStruct((N, H), jnp.float32),
            jax.ShapeDtypeStruct((N, 1), jnp.float32),
        ],
    )(degp, xw)


def _combine(p_ref, u_ref, dinv_ref, b_ref, g_ref, be_ref):
    dv = dinv_ref[...]
    p = p_ref[...]
    agg = p[:, 0:H] + p[:, H:NC * H] + u_ref[...]
    h = dv * agg + b_ref[...]
    return jnp.maximum(h * (g_ref[...] * _BN_S) + be_ref[...], 0.0)


def _t2_body(p_ref, u_ref, dinv_ref, b_ref, g_ref, be_ref, w2_ref, u2_ref):
    h = _combine(p_ref, u_ref, dinv_ref, b_ref, g_ref, be_ref)
    u2_ref[...] = jnp.dot(h, w2_ref[...],
                          preferred_element_type=jnp.float32) * dinv_ref[...]


def _t2(p1, u1, dinv, b1, g1, be1, w2):
    return pl.pallas_call(
        _t2_body,
        out_shape=jax.ShapeDtypeStruct((N, H), jnp.float32),
    )(p1, u1, dinv, b1, g1, be1, w2)


def _t3_body(p_ref, u_ref, dinv_ref, b_ref, g_ref, be_ref, batch_ref,
             lw1_ref, lb1_ref, lw2_ref, lb2_ref, y_ref):
    h = _combine(p_ref, u_ref, dinv_ref, b_ref, g_ref, be_ref)
    bb = batch_ref[...]
    iota = lax.broadcasted_iota(jnp.int32, (G, N), 0)
    oh = (iota == bb).astype(jnp.float32)
    pool = jnp.dot(oh, h, preferred_element_type=jnp.float32)
    cnt = jnp.sum(oh, axis=1, keepdims=True)
    mean = pool / jnp.maximum(cnt, 1.0)
    t = jnp.maximum(
        jnp.dot(mean, lw1_ref[...],
                preferred_element_type=jnp.float32) + lb1_ref[...], 0.0)
    y_ref[...] = jnp.dot(t, lw2_ref[...],
                         preferred_element_type=jnp.float32) + lb2_ref[...]


def _t3(p2, u2, dinv, b2, g2, be2, batch2d, lw1, lb1, lw2, lb2):
    return pl.pallas_call(
        _t3_body,
        out_shape=jax.ShapeDtypeStruct((G, 2), jnp.float32),
    )(p2, u2, dinv, b2, g2, be2, batch2d, lw1, lb1, lw2, lb2)


def kernel(x, edge_index, batch, W1, b1, g1, be1, W2, b2, g2, be2,
           lW1, lb1, lW2, lb2):
    e3d = edge_index.reshape(2, NCHUNK, CH)

    xw = _t1a(x, W1)
    degp = _sc_degree(e3d)
    u1, dinv = _t1b(degp, xw)
    p1 = _sc_agg(e3d, u1)
    u2 = _t2(p1, u1, dinv, b1.reshape(1, H), g1.reshape(1, H),
             be1.reshape(1, H), W2)
    p2 = _sc_agg(e3d, u2)
    y = _t3(p2, u2, dinv, b2.reshape(1, H), g2.reshape(1, H),
            be2.reshape(1, H), batch.reshape(1, N), lW1,
            lb1.reshape(1, H // 2), lW2, lb2.reshape(1, 2))
    return y


# speedup vs baseline: 55.6902x; 1.0537x over previous
import functools

import jax
import jax.numpy as jnp
from jax import lax
from jax.experimental import pallas as pl
from jax.experimental.pallas import tpu as pltpu
from jax.experimental.pallas import tpu_sc as plsc

N = 10000
E = 320000
F = 128
H = 64
G = 128
BN_EPS = 1e-5

NC = 2
NS = 16
LANES = 16
NW = NC * NS
CH = 128
NCHUNK = E // CH
CPW = NCHUNK // NW
XTRA = NCHUNK - CPW * NW
DW = 128
RPT = N // NS
ZR = 125
NBUF = 6


def _sc_mesh():
    return plsc.VectorSubcoreMesh(core_axis_name="c", subcore_axis_name="s",
                                  num_cores=NC, num_subcores=NS)


@functools.partial(
    pl.kernel,
    out_type=jax.ShapeDtypeStruct((NW, N), jnp.float32),
    mesh=_sc_mesh(),
    scratch_types=[
        pltpu.VMEM((CPW + 1, CH), jnp.int32),
        pltpu.VMEM((N,), jnp.float32),
    ],
    compiler_params=pltpu.CompilerParams(use_tc_tiling_on_sc=False,
                                         needs_layout_passes=False),
)
def _sc_degree(e_hbm, out_hbm, didx, degv):
    cid = lax.axis_index("c")
    sid = lax.axis_index("s")
    wid = sid * NC + cid

    pltpu.sync_copy(e_hbm.at[1, pl.ds(wid * CPW, CPW)],
                    didx.at[pl.ds(0, CPW)])

    @pl.when(wid < XTRA)
    def _():
        pltpu.sync_copy(e_hbm.at[1, pl.ds(NW * CPW + wid, 1)],
                        didx.at[pl.ds(CPW, 1)])

    def fill_zeros(i, _):
        degv[pl.ds(i * LANES, LANES)] = jnp.zeros((LANES,), jnp.float32)
        return 0

    lax.fori_loop(0, N // LANES, fill_zeros, 0)

    ones = jnp.ones((LANES,), jnp.float32)

    def body(r, _):
        for k in range(CH // LANES):
            idx = didx[r, pl.ds(k * LANES, LANES)]
            plsc.addupdate_scatter(degv, [idx], ones)
        return 0

    lax.fori_loop(0, CPW, body, 0)

    @pl.when(wid < XTRA)
    def _():
        for k in range(CH // LANES):
            idx = didx[CPW, pl.ds(k * LANES, LANES)]
            plsc.addupdate_scatter(degv, [idx], ones)

    pltpu.sync_copy(degv, out_hbm.at[wid])


@functools.partial(
    pl.kernel,
    out_type=jax.ShapeDtypeStruct((N, NC * H), jnp.float32),
    mesh=_sc_mesh(),
    scratch_types=[
        pltpu.VMEM((CPW + 1, CH), jnp.int32),
        pltpu.VMEM((CPW + 1, CH), jnp.int32),
        [pltpu.VMEM((CH, H), jnp.float32)] * NBUF,
        pltpu.VMEM((ZR, H), jnp.float32),
        pltpu.VMEM_SHARED((N, H), jnp.float32),
        [pltpu.SemaphoreType.DMA] * NBUF,
        [pltpu.SemaphoreType.DMA] * NBUF,
    ],
    compiler_params=pltpu.CompilerParams(use_tc_tiling_on_sc=False,
                                         needs_layout_passes=False),
)
def _sc_agg(e_hbm, u_hbm, out_hbm,
            sidx, didx, rows, zbuf, agg_sh, gs, ss):
    cid = lax.axis_index("c")
    sid = lax.axis_index("s")
    wid = sid * NC + cid

    cps = pltpu.async_copy(e_hbm.at[0, pl.ds(wid * CPW, CPW)],
                           sidx.at[pl.ds(0, CPW)], gs[0])
    cpd = pltpu.async_copy(e_hbm.at[1, pl.ds(wid * CPW, CPW)],
                           didx.at[pl.ds(0, CPW)], gs[1])

    @pl.when(wid < XTRA)
    def _():
        pltpu.sync_copy(e_hbm.at[0, pl.ds(NW * CPW + wid, 1)],
                        sidx.at[pl.ds(CPW, 1)])
        pltpu.sync_copy(e_hbm.at[1, pl.ds(NW * CPW + wid, 1)],
                        didx.at[pl.ds(CPW, 1)])

    def fill_zeros(i, _):
        for k in range(H // LANES):
            zbuf[i, pl.ds(k * LANES, LANES)] = jnp.zeros((LANES,), jnp.float32)
        return 0

    lax.fori_loop(0, ZR, fill_zeros, 0)

    for z in range(RPT // ZR):
        pltpu.sync_copy(zbuf, agg_sh.at[pl.ds(sid * RPT + z * ZR, ZR)])
    cps.wait()
    cpd.wait()

    def dbl(r, _):
        for k in range(CH // LANES):
            s = sidx[r, pl.ds(k * LANES, LANES)]
            sidx[r, pl.ds(k * LANES, LANES)] = s + s
        return 0

    lax.fori_loop(0, CPW + 1, dbl, 0)
    plsc.subcore_barrier()

    for b in range(NBUF):
        pltpu.async_copy(u_hbm.at[sidx.at[b]], rows[b], gs[b])

    def body(j2, _):
        for b in range(NBUF):
            c = NBUF * j2 + b
            pltpu.make_async_copy(u_hbm.at[sidx.at[c]], rows[b], gs[b]).wait()
            pltpu.async_copy(rows[b], agg_sh.at[didx.at[c]], ss[b], add=True)
        for b in range(NBUF):
            cp = jnp.minimum(NBUF * j2 + NBUF + b, CPW - 1)
            pltpu.make_async_copy(rows[b], agg_sh.at[didx.at[0]], ss[b]).wait()
            pltpu.async_copy(u_hbm.at[sidx.at[cp]], rows[b], gs[b])
        return 0

    lax.fori_loop(0, CPW // NBUF, body, 0)
    for b in range(NBUF):
        c = (CPW // NBUF) * NBUF + b
        pltpu.make_async_copy(
            u_hbm.at[sidx.at[min(c, CPW - 1)]], rows[b], gs[b]).wait()
        if c < CPW:
            pltpu.sync_copy(rows[b], agg_sh.at[didx.at[c]], add=True)

    @pl.when(wid < XTRA)
    def _():
        pltpu.sync_copy(u_hbm.at[sidx.at[CPW]], rows[NBUF - 1])
        pltpu.sync_copy(rows[NBUF - 1], agg_sh.at[didx.at[CPW]], add=True)

    plsc.subcore_barrier()
    pltpu.sync_copy(agg_sh.at[pl.ds(sid * RPT, RPT)],
                    out_hbm.at[pl.ds(sid * RPT, RPT), pl.ds(cid * H, H)])


_BN_S = 1.0 / (1.0 + BN_EPS) ** 0.5


def _t1a_body(x_ref, w1_ref, xw_ref):
    xw_ref[...] = jnp.dot(x_ref[...], w1_ref[...],
                          preferred_element_type=jnp.float32)


def _t1a(x, w1):
    return pl.pallas_call(
        _t1a_body,
        out_shape=jax.ShapeDtypeStruct((N, NC * H), jnp.float32),
    )(x, w1)


def _t1b_body(deg_ref, xw_ref, u1_ref, dinv_ref):
    d = deg_ref[...]
    dsum = lax.dot_general(d, jnp.ones((NW, 1), jnp.float32),
                           (((0,), (0,)), ((), ())),
                           preferred_element_type=jnp.float32)
    dv = lax.rsqrt(dsum + 1.0)
    u1_ref[...] = xw_ref[...] * dv
    dinv_ref[...] = dv


def _t1b(degp, xw):
    return pl.pallas_call(
        _t1b_body,
        out_shape=[
            jax.ShapeDtypeStruct((N, NC * H), jnp.float32),
            jax.ShapeDtypeStruct((N, 1), jnp.float32),
        ],
    )(degp, xw)


def _combine(p_ref, u_ref, dinv_ref, b_ref, g_ref, be_ref):
    dv = dinv_ref[...]
    p = p_ref[...]
    u = u_ref[...]
    agg = p[:, 0:H] + p[:, H:NC * H] + u[:, 0:H]
    h = dv * agg + b_ref[...]
    return jnp.maximum(h * (g_ref[...] * _BN_S) + be_ref[...], 0.0)


def _t2_body(p_ref, u_ref, dinv_ref, b_ref, g_ref, be_ref, w2_ref, u2_ref):
    h = _combine(p_ref, u_ref, dinv_ref, b_ref, g_ref, be_ref)
    u2_ref[...] = jnp.dot(h, w2_ref[...],
                          preferred_element_type=jnp.float32) * dinv_ref[...]


def _t2(p1, u1, dinv, b1, g1, be1, w2pad):
    return pl.pallas_call(
        _t2_body,
        out_shape=jax.ShapeDtypeStruct((N, NC * H), jnp.float32),
    )(p1, u1, dinv, b1, g1, be1, w2pad)


def _t3_body(p_ref, u_ref, dinv_ref, b_ref, g_ref, be_ref, batch_ref,
             lw1_ref, lb1_ref, lw2_ref, lb2_ref, y_ref):
    h = _combine(p_ref, u_ref, dinv_ref, b_ref, g_ref, be_ref)
    bb = batch_ref[...]
    iota = lax.broadcasted_iota(jnp.int32, (G, N), 0)
    oh = (iota == bb).astype(jnp.float32)
    pool = jnp.dot(oh, h, preferred_element_type=jnp.float32)
    cnt = jnp.sum(oh, axis=1, keepdims=True)
    mean = pool / jnp.maximum(cnt, 1.0)
    t = jnp.maximum(
        jnp.dot(mean, lw1_ref[...],
                preferred_element_type=jnp.float32) + lb1_ref[...], 0.0)
    y_ref[...] = jnp.dot(t, lw2_ref[...],
                         preferred_element_type=jnp.float32) + lb2_ref[...]


def _t3(p2, u2, dinv, b2, g2, be2, batch2d, lw1, lb1, lw2, lb2):
    return pl.pallas_call(
        _t3_body,
        out_shape=jax.ShapeDtypeStruct((G, 2), jnp.float32),
    )(p2, u2, dinv, b2, g2, be2, batch2d, lw1, lb1, lw2, lb2)


def kernel(x, edge_index, batch, W1, b1, g1, be1, W2, b2, g2, be2,
           lW1, lb1, lW2, lb2):
    e3d = edge_index.reshape(2, NCHUNK, CH)
    w1pad = jnp.pad(W1, ((0, 0), (0, H)))
    w2pad = jnp.pad(W2, ((0, 0), (0, H)))

    xw = _t1a(x, w1pad)
    degp = _sc_degree(e3d)
    u1, dinv = _t1b(degp, xw)
    p1 = _sc_agg(e3d, u1.reshape(2 * N, H))
    u2 = _t2(p1, u1, dinv, b1.reshape(1, H), g1.reshape(1, H),
             be1.reshape(1, H), w2pad)
    p2 = _sc_agg(e3d, u2.reshape(2 * N, H))
    y = _t3(p2, u2, dinv, b2.reshape(1, H), g2.reshape(1, H),
            be2.reshape(1, H), batch.reshape(1, N), lW1,
            lb1.reshape(1, H // 2), lW2, lb2.reshape(1, 2))
    return y


# speedup vs baseline: 57.3443x; 1.0297x over previous
import functools

import jax
import jax.numpy as jnp
from jax import lax
from jax.experimental import pallas as pl
from jax.experimental.pallas import tpu as pltpu
from jax.experimental.pallas import tpu_sc as plsc

N = 10000
E = 320000
F = 128
H = 64
G = 128
BN_EPS = 1e-5

NC = 2
NS = 16
LANES = 16
NW = NC * NS
CH = 128
NCHUNK = E // CH
CPW = NCHUNK // NW
XTRA = NCHUNK - CPW * NW
DW = 128
RPT = N // NS
ZR = 125
NBUF = 6


def _sc_mesh():
    return plsc.VectorSubcoreMesh(core_axis_name="c", subcore_axis_name="s",
                                  num_cores=NC, num_subcores=NS)


EPW = CPW * CH


@functools.partial(
    pl.kernel,
    out_type=jax.ShapeDtypeStruct((NW, N), jnp.float32),
    mesh=_sc_mesh(),
    scratch_types=[
        pltpu.VMEM((2, EPW + CH), jnp.int32),
        pltpu.VMEM((N,), jnp.float32),
    ],
    compiler_params=pltpu.CompilerParams(needs_layout_passes=False),
)
def _sc_degree(e_hbm, out_hbm, didx, degv):
    cid = lax.axis_index("c")
    sid = lax.axis_index("s")
    wid = sid * NC + cid

    pltpu.sync_copy(e_hbm.at[:, pl.ds(wid * EPW, EPW)],
                    didx.at[:, pl.ds(0, EPW)])

    @pl.when(wid < XTRA)
    def _():
        pltpu.sync_copy(e_hbm.at[:, pl.ds(NW * EPW + wid * CH, CH)],
                        didx.at[:, pl.ds(EPW, CH)])

    def fill_zeros(i, _):
        degv[pl.ds(i * LANES, LANES)] = jnp.zeros((LANES,), jnp.float32)
        return 0

    lax.fori_loop(0, N // LANES, fill_zeros, 0)

    ones = jnp.ones((LANES,), jnp.float32)

    def body(r, _):
        for k in range(8):
            idx = didx[1, pl.ds(r * 8 * LANES + k * LANES, LANES)]
            plsc.addupdate_scatter(degv, [idx], ones)
        return 0

    lax.fori_loop(0, EPW // (8 * LANES), body, 0)

    @pl.when(wid < XTRA)
    def _():
        for k in range(CH // LANES):
            idx = didx[1, pl.ds(EPW + k * LANES, LANES)]
            plsc.addupdate_scatter(degv, [idx], ones)

    pltpu.sync_copy(degv, out_hbm.at[wid])


@functools.partial(
    pl.kernel,
    out_type=jax.ShapeDtypeStruct((N, NC * H), jnp.float32),
    mesh=_sc_mesh(),
    scratch_types=[
        pltpu.VMEM((CPW + 1, CH), jnp.int32),
        pltpu.VMEM((CPW + 1, CH), jnp.int32),
        [pltpu.VMEM((CH, H), jnp.float32)] * NBUF,
        pltpu.VMEM((ZR, H), jnp.float32),
        pltpu.VMEM_SHARED((N, H), jnp.float32),
        [pltpu.SemaphoreType.DMA] * NBUF,
        [pltpu.SemaphoreType.DMA] * NBUF,
    ],
    compiler_params=pltpu.CompilerParams(use_tc_tiling_on_sc=False,
                                         needs_layout_passes=False),
)
def _sc_agg(e_hbm, u_hbm, out_hbm,
            sidx, didx, rows, zbuf, agg_sh, gs, ss):
    cid = lax.axis_index("c")
    sid = lax.axis_index("s")
    wid = sid * NC + cid

    cps = pltpu.async_copy(e_hbm.at[0, pl.ds(wid * CPW, CPW)],
                           sidx.at[pl.ds(0, CPW)], gs[0])
    cpd = pltpu.async_copy(e_hbm.at[1, pl.ds(wid * CPW, CPW)],
                           didx.at[pl.ds(0, CPW)], gs[1])

    @pl.when(wid < XTRA)
    def _():
        pltpu.sync_copy(e_hbm.at[0, pl.ds(NW * CPW + wid, 1)],
                        sidx.at[pl.ds(CPW, 1)])
        pltpu.sync_copy(e_hbm.at[1, pl.ds(NW * CPW + wid, 1)],
                        didx.at[pl.ds(CPW, 1)])

    def fill_zeros(i, _):
        for k in range(H // LANES):
            zbuf[i, pl.ds(k * LANES, LANES)] = jnp.zeros((LANES,), jnp.float32)
        return 0

    lax.fori_loop(0, ZR, fill_zeros, 0)

    for z in range(RPT // ZR):
        pltpu.sync_copy(zbuf, agg_sh.at[pl.ds(sid * RPT + z * ZR, ZR)])
    cps.wait()
    cpd.wait()

    def dbl(r, _):
        for k in range(CH // LANES):
            s = sidx[r, pl.ds(k * LANES, LANES)]
            sidx[r, pl.ds(k * LANES, LANES)] = s + s
        return 0

    lax.fori_loop(0, CPW + 1, dbl, 0)
    plsc.subcore_barrier()

    for b in range(NBUF):
        pltpu.async_copy(u_hbm.at[sidx.at[b]], rows[b], gs[b])

    def body(j2, _):
        for b in range(NBUF):
            c = NBUF * j2 + b
            pltpu.make_async_copy(u_hbm.at[sidx.at[c]], rows[b], gs[b]).wait()
            pltpu.async_copy(rows[b], agg_sh.at[didx.at[c]], ss[b], add=True)
        for b in range(NBUF):
            cp = jnp.minimum(NBUF * j2 + NBUF + b, CPW - 1)
            pltpu.make_async_copy(rows[b], agg_sh.at[didx.at[0]], ss[b]).wait()
            pltpu.async_copy(u_hbm.at[sidx.at[cp]], rows[b], gs[b])
        return 0

    lax.fori_loop(0, CPW // NBUF, body, 0)
    for b in range(NBUF):
        c = (CPW // NBUF) * NBUF + b
        pltpu.make_async_copy(
            u_hbm.at[sidx.at[min(c, CPW - 1)]], rows[b], gs[b]).wait()
        if c < CPW:
            pltpu.sync_copy(rows[b], agg_sh.at[didx.at[c]], add=True)

    @pl.when(wid < XTRA)
    def _():
        pltpu.sync_copy(u_hbm.at[sidx.at[CPW]], rows[NBUF - 1])
        pltpu.sync_copy(rows[NBUF - 1], agg_sh.at[didx.at[CPW]], add=True)

    plsc.subcore_barrier()
    pltpu.sync_copy(agg_sh.at[pl.ds(sid * RPT, RPT)],
                    out_hbm.at[pl.ds(sid * RPT, RPT), pl.ds(cid * H, H)])


_BN_S = 1.0 / (1.0 + BN_EPS) ** 0.5


def _t1a_body(x_ref, w1_ref, xw_ref):
    xw_ref[...] = jnp.dot(x_ref[...], w1_ref[...],
                          preferred_element_type=jnp.float32)


def _t1a(x, w1):
    return pl.pallas_call(
        _t1a_body,
        out_shape=jax.ShapeDtypeStruct((N, NC * H), jnp.float32),
    )(x, w1)


def _t1b_body(deg_ref, xw_ref, u1_ref, dinv_ref):
    d = deg_ref[...]
    dsum = lax.dot_general(d, jnp.ones((NW, 1), jnp.float32),
                           (((0,), (0,)), ((), ())),
                           preferred_element_type=jnp.float32)
    dv = lax.rsqrt(dsum + 1.0)
    u1_ref[...] = xw_ref[...] * dv
    dinv_ref[...] = dv


def _t1b(degp, xw):
    return pl.pallas_call(
        _t1b_body,
        out_shape=[
            jax.ShapeDtypeStruct((N, NC * H), jnp.float32),
            jax.ShapeDtypeStruct((N, 1), jnp.float32),
        ],
    )(degp, xw)


def _combine(p_ref, u_ref, dinv_ref, b_ref, g_ref, be_ref):
    dv = dinv_ref[...]
    p = p_ref[...]
    u = u_ref[...]
    agg = p[:, 0:H] + p[:, H:NC * H] + u[:, 0:H]
    h = dv * agg + b_ref[...]
    return jnp.maximum(h * (g_ref[...] * _BN_S) + be_ref[...], 0.0)


def _t2_body(p_ref, u_ref, dinv_ref, b_ref, g_ref, be_ref, w2_ref, u2_ref):
    h = _combine(p_ref, u_ref, dinv_ref, b_ref, g_ref, be_ref)
    u2_ref[...] = jnp.dot(h, w2_ref[...],
                          preferred_element_type=jnp.float32) * dinv_ref[...]


def _t2(p1, u1, dinv, b1, g1, be1, w2pad):
    return pl.pallas_call(
        _t2_body,
        out_shape=jax.ShapeDtypeStruct((N, NC * H), jnp.float32),
    )(p1, u1, dinv, b1, g1, be1, w2pad)


def _t3_body(p_ref, u_ref, dinv_ref, b_ref, g_ref, be_ref, batch_ref,
             lw1_ref, lb1_ref, lw2_ref, lb2_ref, y_ref):
    h = _combine(p_ref, u_ref, dinv_ref, b_ref, g_ref, be_ref)
    bb = batch_ref[...]
    iota = lax.broadcasted_iota(jnp.int32, (G, N), 0)
    oh = (iota == bb).astype(jnp.float32)
    pool = jnp.dot(oh, h, preferred_element_type=jnp.float32)
    cnt = jnp.sum(oh, axis=1, keepdims=True)
    mean = pool / jnp.maximum(cnt, 1.0)
    t = jnp.maximum(
        jnp.dot(mean, lw1_ref[...],
                preferred_element_type=jnp.float32) + lb1_ref[...], 0.0)
    y_ref[...] = jnp.dot(t, lw2_ref[...],
                         preferred_element_type=jnp.float32) + lb2_ref[...]


def _t3(p2, u2, dinv, b2, g2, be2, batch2d, lw1, lb1, lw2, lb2):
    return pl.pallas_call(
        _t3_body,
        out_shape=jax.ShapeDtypeStruct((G, 2), jnp.float32),
    )(p2, u2, dinv, b2, g2, be2, batch2d, lw1, lb1, lw2, lb2)


def kernel(x, edge_index, batch, W1, b1, g1, be1, W2, b2, g2, be2,
           lW1, lb1, lW2, lb2):
    e3d = edge_index.reshape(2, NCHUNK, CH)
    w1pad = jnp.pad(W1, ((0, 0), (0, H)))
    w2pad = jnp.pad(W2, ((0, 0), (0, H)))

    xw = _t1a(x, w1pad)
    degp = _sc_degree(edge_index)
    u1, dinv = _t1b(degp, xw)
    p1 = _sc_agg(e3d, u1.reshape(2 * N, H))
    u2 = _t2(p1, u1, dinv, b1.reshape(1, H), g1.reshape(1, H),
             be1.reshape(1, H), w2pad)
    p2 = _sc_agg(e3d, u2.reshape(2 * N, H))
    y = _t3(p2, u2, dinv, b2.reshape(1, H), g2.reshape(1, H),
            be2.reshape(1, H), batch.reshape(1, N), lW1,
            lb1.reshape(1, H // 2), lW2, lb2.reshape(1, 2))
    return y


# speedup vs baseline: 58.3955x; 1.0183x over previous
import functools

import jax
import jax.numpy as jnp
from jax import lax
from jax.experimental import pallas as pl
from jax.experimental.pallas import tpu as pltpu
from jax.experimental.pallas import tpu_sc as plsc

N = 10000
E = 320000
F = 128
H = 64
G = 128
BN_EPS = 1e-5

NC = 2
NS = 16
LANES = 16
NW = NC * NS
CH = 128
NCHUNK = E // CH
CPW = NCHUNK // NW
XTRA = NCHUNK - CPW * NW
DW = 128
RPT = N // NS
ZR = 125
NBUF = 6


def _sc_mesh():
    return plsc.VectorSubcoreMesh(core_axis_name="c", subcore_axis_name="s",
                                  num_cores=NC, num_subcores=NS)


EPW = CPW * CH


@functools.partial(
    pl.kernel,
    out_type=jax.ShapeDtypeStruct((NW, N), jnp.float32),
    mesh=_sc_mesh(),
    scratch_types=[
        pltpu.VMEM((2, EPW + CH), jnp.int32),
        pltpu.VMEM((N,), jnp.float32),
    ],
    compiler_params=pltpu.CompilerParams(needs_layout_passes=False),
)
def _sc_degree(e_hbm, out_hbm, didx, degv):
    cid = lax.axis_index("c")
    sid = lax.axis_index("s")
    wid = sid * NC + cid

    pltpu.sync_copy(e_hbm.at[:, pl.ds(wid * EPW, EPW)],
                    didx.at[:, pl.ds(0, EPW)])

    @pl.when(wid < XTRA)
    def _():
        pltpu.sync_copy(e_hbm.at[:, pl.ds(NW * EPW + wid * CH, CH)],
                        didx.at[:, pl.ds(EPW, CH)])

    def fill_zeros(i, _):
        degv[pl.ds(i * LANES, LANES)] = jnp.zeros((LANES,), jnp.float32)
        return 0

    lax.fori_loop(0, N // LANES, fill_zeros, 0)

    ones = jnp.ones((LANES,), jnp.float32)

    def body(r, _):
        for k in range(8):
            idx = didx[1, pl.ds(r * 8 * LANES + k * LANES, LANES)]
            plsc.addupdate_scatter(degv, [idx], ones)
        return 0

    lax.fori_loop(0, EPW // (8 * LANES), body, 0)

    @pl.when(wid < XTRA)
    def _():
        for k in range(CH // LANES):
            idx = didx[1, pl.ds(EPW + k * LANES, LANES)]
            plsc.addupdate_scatter(degv, [idx], ones)

    pltpu.sync_copy(degv, out_hbm.at[wid])


@functools.partial(
    pl.kernel,
    out_type=jax.ShapeDtypeStruct((N, NC * H), jnp.float32),
    mesh=_sc_mesh(),
    scratch_types=[
        pltpu.VMEM((CPW + 1, CH), jnp.int32),
        pltpu.VMEM((CPW + 1, CH), jnp.int32),
        [pltpu.VMEM((CH, H), jnp.float32)] * NBUF,
        pltpu.VMEM((ZR, H), jnp.float32),
        pltpu.VMEM_SHARED((N, H), jnp.float32),
        [pltpu.SemaphoreType.DMA] * NBUF,
        [pltpu.SemaphoreType.DMA] * NBUF,
    ],
    compiler_params=pltpu.CompilerParams(use_tc_tiling_on_sc=False,
                                         needs_layout_passes=False),
)
def _sc_agg(e_hbm, u_hbm, out_hbm,
            sidx, didx, rows, zbuf, agg_sh, gs, ss):
    cid = lax.axis_index("c")
    sid = lax.axis_index("s")
    wid = sid * NC + cid

    cps = pltpu.async_copy(e_hbm.at[0, pl.ds(wid * CPW, CPW)],
                           sidx.at[pl.ds(0, CPW)], gs[0])
    cpd = pltpu.async_copy(e_hbm.at[1, pl.ds(wid * CPW, CPW)],
                           didx.at[pl.ds(0, CPW)], gs[1])

    @pl.when(wid < XTRA)
    def _():
        pltpu.sync_copy(e_hbm.at[0, pl.ds(NW * CPW + wid, 1)],
                        sidx.at[pl.ds(CPW, 1)])
        pltpu.sync_copy(e_hbm.at[1, pl.ds(NW * CPW + wid, 1)],
                        didx.at[pl.ds(CPW, 1)])

    def fill_zeros(i, _):
        for k in range(H // LANES):
            zbuf[i, pl.ds(k * LANES, LANES)] = jnp.zeros((LANES,), jnp.float32)
        return 0

    lax.fori_loop(0, ZR, fill_zeros, 0)

    for z in range(RPT // ZR):
        pltpu.sync_copy(zbuf, agg_sh.at[pl.ds(sid * RPT + z * ZR, ZR)])
    cps.wait()
    cpd.wait()

    def dbl(r, _):
        for k in range(CH // LANES):
            s = sidx[r, pl.ds(k * LANES, LANES)]
            sidx[r, pl.ds(k * LANES, LANES)] = s + s
        return 0

    lax.fori_loop(0, CPW + 1, dbl, 0)
    plsc.subcore_barrier()

    for b in range(NBUF):
        pltpu.async_copy(u_hbm.at[sidx.at[b]], rows[b], gs[b])

    def body(j2, _):
        for b in range(NBUF):
            c = NBUF * j2 + b
            pltpu.make_async_copy(u_hbm.at[sidx.at[c]], rows[b], gs[b]).wait()
            pltpu.async_copy(rows[b], agg_sh.at[didx.at[c]], ss[b], add=True)
        for b in range(NBUF):
            cp = jnp.minimum(NBUF * j2 + NBUF + b, CPW - 1)
            pltpu.make_async_copy(rows[b], agg_sh.at[didx.at[0]], ss[b]).wait()
            pltpu.async_copy(u_hbm.at[sidx.at[cp]], rows[b], gs[b])
        return 0

    lax.fori_loop(0, CPW // NBUF, body, 0)
    for b in range(NBUF):
        c = (CPW // NBUF) * NBUF + b
        pltpu.make_async_copy(
            u_hbm.at[sidx.at[min(c, CPW - 1)]], rows[b], gs[b]).wait()
        if c < CPW:
            pltpu.sync_copy(rows[b], agg_sh.at[didx.at[c]], add=True)

    @pl.when(wid < XTRA)
    def _():
        pltpu.sync_copy(u_hbm.at[sidx.at[CPW]], rows[NBUF - 1])
        pltpu.sync_copy(rows[NBUF - 1], agg_sh.at[didx.at[CPW]], add=True)

    plsc.subcore_barrier()
    pltpu.sync_copy(agg_sh.at[pl.ds(sid * RPT, RPT)],
                    out_hbm.at[pl.ds(sid * RPT, RPT), pl.ds(cid * H, H)])


_BN_S = 1.0 / (1.0 + BN_EPS) ** 0.5


def _t1a_body(x_ref, w1_ref, xw_ref):
    xw_ref[...] = jnp.dot(x_ref[...], w1_ref[...],
                          preferred_element_type=jnp.float32)


def _t1a(x, w1):
    return pl.pallas_call(
        _t1a_body,
        out_shape=jax.ShapeDtypeStruct((N, NC * H), jnp.float32),
    )(x, w1)


def _dv_col(dv):
    lanes = lax.broadcasted_iota(jnp.int32, (N, NC * H), 1)
    return jnp.where(lanes == H, dv, 0.0)


def _t1b_body(deg_ref, xw_ref, u1_ref):
    d = deg_ref[...]
    dsum = lax.dot_general(d, jnp.ones((NW, 1), jnp.float32),
                           (((0,), (0,)), ((), ())),
                           preferred_element_type=jnp.float32)
    dv = lax.rsqrt(dsum + 1.0)
    u1_ref[...] = xw_ref[...] * dv + _dv_col(dv)


def _t1b(degp, xw):
    return pl.pallas_call(
        _t1b_body,
        out_shape=jax.ShapeDtypeStruct((N, NC * H), jnp.float32),
    )(degp, xw)


def _combine(p_ref, u_ref, b_ref, g_ref, be_ref):
    p = p_ref[...]
    u = u_ref[...]
    dv = u[:, H:H + 1]
    agg = p[:, 0:H] + p[:, H:NC * H] + u[:, 0:H]
    h = dv * agg + b_ref[...]
    return jnp.maximum(h * (g_ref[...] * _BN_S) + be_ref[...], 0.0), dv


def _t2_body(p_ref, u_ref, b_ref, g_ref, be_ref, w2_ref, u2_ref):
    h, dv = _combine(p_ref, u_ref, b_ref, g_ref, be_ref)
    u2_ref[...] = jnp.dot(h, w2_ref[...],
                          preferred_element_type=jnp.float32) * dv + _dv_col(dv)


def _t2(p1, u1, b1, g1, be1, w2pad):
    return pl.pallas_call(
        _t2_body,
        out_shape=jax.ShapeDtypeStruct((N, NC * H), jnp.float32),
    )(p1, u1, b1, g1, be1, w2pad)


def _t3_body(p_ref, u_ref, b_ref, g_ref, be_ref, batch_ref,
             lw1_ref, lb1_ref, lw2_ref, lb2_ref, y_ref):
    h, _ = _combine(p_ref, u_ref, b_ref, g_ref, be_ref)
    bb = batch_ref[...]
    iota = lax.broadcasted_iota(jnp.int32, (G, N), 0)
    oh = (iota == bb).astype(jnp.float32)
    pool = jnp.dot(oh, h, preferred_element_type=jnp.float32)
    cnt = jnp.sum(oh, axis=1, keepdims=True)
    mean = pool / jnp.maximum(cnt, 1.0)
    t = jnp.maximum(
        jnp.dot(mean, lw1_ref[...],
                preferred_element_type=jnp.float32) + lb1_ref[...], 0.0)
    y_ref[...] = jnp.dot(t, lw2_ref[...],
                         preferred_element_type=jnp.float32) + lb2_ref[...]


def _t3(p2, u2, b2, g2, be2, batch2d, lw1, lb1, lw2, lb2):
    return pl.pallas_call(
        _t3_body,
        out_shape=jax.ShapeDtypeStruct((G, 2), jnp.float32),
    )(p2, u2, b2, g2, be2, batch2d, lw1, lb1, lw2, lb2)


def kernel(x, edge_index, batch, W1, b1, g1, be1, W2, b2, g2, be2,
           lW1, lb1, lW2, lb2):
    e3d = edge_index.reshape(2, NCHUNK, CH)
    w1pad = jnp.pad(W1, ((0, 0), (0, H)))
    w2pad = jnp.pad(W2, ((0, 0), (0, H)))

    xw = _t1a(x, w1pad)
    degp = _sc_degree(edge_index)
    u1 = _t1b(degp, xw)
    p1 = _sc_agg(e3d, u1.reshape(2 * N, H))
    u2 = _t2(p1, u1, b1.reshape(1, H), g1.reshape(1, H),
             be1.reshape(1, H), w2pad)
    p2 = _sc_agg(e3d, u2.reshape(2 * N, H))
    y = _t3(p2, u2, b2.reshape(1, H), g2.reshape(1, H),
            be2.reshape(1, H), batch.reshape(1, N), lW1,
            lb1.reshape(1, H // 2), lW2, lb2.reshape(1, 2))
    return y
